# baseline (speedup 1.0000x reference)
import functools

import jax
import jax.numpy as jnp
from jax import lax
from jax.experimental import pallas as pl
from jax.experimental.pallas import tpu as pltpu
from jax.experimental.pallas import tpu_sc as plsc

B = 16384
D = 64
DP = 2 * D
L = 16
NC = 2
NS = 16
NW = NC * NS
BPW = B // NW
CHUNK = 128
NCHUNK = BPW // CHUNK
HALF = 256
NGROUP = HALF // L

_mesh = plsc.VectorSubcoreMesh(core_axis_name="c", subcore_axis_name="s")
_params = pltpu.CompilerParams(
    needs_layout_passes=False,
    has_side_effects=pltpu.SideEffectType.PURE,
)


def _gather_body(id_hbm, tbl_hbm, rows_hbm, idx_v, q_v, blk_v, row_v, sem):
    wid = lax.axis_index("s") * NC + lax.axis_index("c")
    base = wid * BPW

    for c in range(NCHUNK):
        pltpu.sync_copy(id_hbm.at[pl.ds(base + c * CHUNK, CHUNK)],
                        idx_v.at[c])
    for c in range(NCHUNK):
        for t in range(CHUNK // L):
            q_v[c, pl.ds(t * L, L)] = (
                lax.shift_right_logical(idx_v[c, pl.ds(t * L, L)], 1))

    for h in range(2):
        copies = [pltpu.async_copy(
            tbl_hbm.at[q_v.at[h * (HALF // CHUNK) + c]],
            blk_v.at[pl.ds(c * CHUNK, CHUNK)], sem)
            for c in range(HALF // CHUNK)]
        for cp in copies:
            cp.wait()

        def row_group(g, carry):
            r0 = g * L
            gpos = h * HALF + r0
            cc = lax.shift_right_logical(gpos, 7)
            oo = lax.bitwise_and(gpos, 127)
            vec = idx_v[cc, pl.ds(oo, L)]
            for rr in range(L):
                r = r0 + rr
                p = (vec[rr] % 2) * D
                rq = lax.shift_right_logical(r, 1)
                ro = (rr % 2) * D
                for j in range(D // L):
                    row_v[rq, pl.ds(ro + j * L, L)] = (
                        blk_v[r, pl.ds(p + j * L, L)])
            return carry

        lax.fori_loop(0, NGROUP, row_group, 0, unroll=False)

        pltpu.sync_copy(
            row_v,
            rows_hbm.at[pl.ds(
                pl.multiple_of((base + h * HALF) // 2, HALF // 2),
                HALF // 2)])


_sc_scratch = [
    pltpu.VMEM((NCHUNK, CHUNK), jnp.int32),
    pltpu.VMEM((NCHUNK, CHUNK), jnp.int32),
    pltpu.VMEM((HALF, DP), jnp.float32),
    pltpu.VMEM((HALF // 2, DP), jnp.float32),
    pltpu.SemaphoreType.DMA,
]


@functools.partial(
    pl.kernel, mesh=_mesh, compiler_params=_params,
    out_type=jax.ShapeDtypeStruct((B // 2, DP), jnp.float32),
    scratch_types=_sc_scratch,
)
def _user_gather(uid_hbm, ut_hbm, uw_hbm, idx_v, q_v, blk_v, row_v, sem):
    _gather_body(uid_hbm, ut_hbm, uw_hbm, idx_v, q_v, blk_v, row_v, sem)


@functools.partial(
    pl.kernel, mesh=_mesh, compiler_params=_params,
    out_type=jax.ShapeDtypeStruct((B // 2, DP), jnp.float32),
    scratch_types=_sc_scratch,
)
def _item_gather(iid_hbm, it_hbm, iw_hbm, idx_v, q_v, blk_v, row_v, sem):
    _gather_body(iid_hbm, it_hbm, iw_hbm, idx_v, q_v, blk_v, row_v, sem)


def _combine_body(u_ref, i_ref, w_ref, b_ref, e_ref, o_ref):
    prod = u_ref[...] * i_ref[...] * w_ref[...]
    bias = b_ref[0, 0]
    e_ref[...] = prod[:, :D].sum(axis=1) + bias
    o_ref[...] = prod[:, D:].sum(axis=1) + bias


_TCBLK = 2048


def _combine(uw, iw, w2, b):
    grid = (B // 2) // _TCBLK
    return pl.pallas_call(
        _combine_body,
        grid=(grid,),
        in_specs=[
            pl.BlockSpec((_TCBLK, DP), lambda g: (g, 0)),
            pl.BlockSpec((_TCBLK, DP), lambda g: (g, 0)),
            pl.BlockSpec((1, DP), lambda g: (0, 0)),
            pl.BlockSpec((1, 1), lambda g: (0, 0), memory_space=pltpu.SMEM),
        ],
        out_specs=[
            pl.BlockSpec((_TCBLK,), lambda g: (g,)),
            pl.BlockSpec((_TCBLK,), lambda g: (g,)),
        ],
        out_shape=[
            jax.ShapeDtypeStruct((B // 2,), jnp.float32),
            jax.ShapeDtypeStruct((B // 2,), jnp.float32),
        ],
    )(uw, iw, w2, b)


def kernel(user_ids, item_ids, user_table, item_table, fc_w, fc_b):
    ut2 = user_table.reshape(user_table.shape[0] // 2, DP)
    it2 = item_table.reshape(item_table.shape[0] // 2, DP)
    uw = _user_gather(user_ids, ut2)
    iw = _item_gather(item_ids, it2)
    w2 = jnp.concatenate([fc_w, fc_w], axis=1)
    b = fc_b.reshape(1, 1)
    ev, od = _combine(uw, iw, w2, b)
    return jnp.stack([ev, od], axis=1).reshape(B)


# speedup vs baseline: 1.5861x; 1.5861x over previous
import functools

import jax
import jax.numpy as jnp
from jax import lax
from jax.experimental import pallas as pl
from jax.experimental.pallas import tpu as pltpu
from jax.experimental.pallas import tpu_sc as plsc

B = 16384
D = 64
L = 16
NC = 2
NS = 16
NW = NC * NS
BPW = B // NW
HALF = 256
NGROUP = HALF // L

_mesh = plsc.VectorSubcoreMesh(core_axis_name="c", subcore_axis_name="s")


@functools.partial(
    pl.kernel,
    mesh=_mesh,
    compiler_params=pltpu.CompilerParams(needs_layout_passes=False),
    out_type=jax.ShapeDtypeStruct((B,), jnp.float32),
    scratch_types=[
        pltpu.VMEM((BPW,), jnp.int32),
        pltpu.VMEM((BPW,), jnp.int32),
        pltpu.VMEM((HALF, D), jnp.float32),
        pltpu.VMEM((HALF, D), jnp.float32),
        pltpu.VMEM((D,), jnp.float32),
        pltpu.VMEM((L,), jnp.float32),
        pltpu.VMEM((BPW,), jnp.float32),
        pltpu.SemaphoreType.DMA,
        pltpu.SemaphoreType.DMA,
    ],
)
def _sc_kernel(uid_hbm, iid_hbm, ut_hbm, it_hbm, w_hbm, b_hbm, out_hbm,
               uidx_v, iidx_v, urows_v, irows_v, w_v, b_v, out_v,
               usem, isem):
    wid = lax.axis_index("s") * NC + lax.axis_index("c")
    base = wid * BPW

    pltpu.sync_copy(uid_hbm.at[pl.ds(base, BPW)], uidx_v)
    pltpu.sync_copy(iid_hbm.at[pl.ds(base, BPW)], iidx_v)
    pltpu.sync_copy(w_hbm, w_v)
    pltpu.sync_copy(b_hbm, b_v)

    wvecs = [w_v[pl.ds(j * L, L)] for j in range(D // L)]
    bvec = b_v[...]
    liota = lax.iota(jnp.int32, L)

    for h in range(2):
        hbase = h * HALF
        copies = []
        for k in range(HALF):
            if k % L == 0:
                uvec = uidx_v[pl.ds(hbase + k, L)]
                ivec = iidx_v[pl.ds(hbase + k, L)]
            u = uvec[k % L]
            i = ivec[k % L]
            copies.append(pltpu.async_copy(
                ut_hbm.at[u], urows_v.at[k], usem))
            copies.append(pltpu.async_copy(
                it_hbm.at[i], irows_v.at[k], isem))
        for cp in copies:
            cp.wait()

        def group_body(g, carry):
            r0 = g * L
            acc = bvec
            for rr in range(L):
                r = r0 + rr
                s = None
                for j in range(D // L):
                    t = (urows_v[r, pl.ds(j * L, L)]
                         * irows_v[r, pl.ds(j * L, L)] * wvecs[j])
                    s = t if s is None else s + t
                acc = jnp.where(liota == rr, acc + jnp.sum(s), acc)
            out_v[pl.ds(hbase + r0, L)] = acc
            return carry

        lax.fori_loop(0, NGROUP, group_body, 0, unroll=False)

    pltpu.sync_copy(out_v, out_hbm.at[pl.ds(base, BPW)])


def kernel(user_ids, item_ids, user_table, item_table, fc_w, fc_b):
    w = fc_w.reshape(D)
    b = jnp.broadcast_to(fc_b.reshape(1), (L,))
    return _sc_kernel(user_ids, item_ids, user_table, item_table, w, b)
